# baseline (speedup 1.0000x reference)
import functools

import jax
import jax.numpy as jnp
from jax import lax
from jax.experimental import pallas as pl
from jax.experimental.pallas import tpu as pltpu
from jax.experimental.pallas import tpu_sc as plsc

_B, _N, _D = 2, 100000, 3
_H = 128
_CD = 128
_R = 32
_V = _R ** 3
_NB = 5

_NW = 32
_BN = 2000
_VR = 512
_NR = _V // (_VR * _NW)
_NREG = _B * _NW * _NR
_SCH = 2000
_GCH = 128
_FB = 2048
_ACC = 4096
_PCAP = 8192
_NP = 110000

_F32 = jnp.float32
_I32 = jnp.int32

_NEG = -3.0e38


def _dot(a, b):
    return jnp.dot(a.astype(jnp.bfloat16), b.astype(jnp.bfloat16),
                   preferred_element_type=_F32)


def _stage_a_body(x_ref, wp_ref, bp_ref, w0_ref, b0_ref, w1_ref, b1_ref,
                  ws_ref, out_ref):
    x = x_ref[...]
    x1 = _dot(x, wp_ref[...]) + bp_ref[...]
    h = _dot(jax.nn.relu(x1), w0_ref[...]) + b0_ref[...]
    dx = _dot(jax.nn.relu(h), w1_ref[...]) + b1_ref[...]
    out_ref[...] = _dot(x1, ws_ref[...]) + dx


def _stage_a(pts_pad, wp, bp, w0, b0, w1, b1, ws):
    nblk = _NP // _BN

    def spec2(a):
        return pl.BlockSpec(a.shape, lambda i: (0, 0))

    return pl.pallas_call(
        _stage_a_body,
        grid=(nblk,),
        in_specs=[
            pl.BlockSpec((_BN, _H), lambda i: (i, 0)),
            spec2(wp), spec2(bp), spec2(w0), spec2(b0), spec2(w1), spec2(b1),
            spec2(ws),
        ],
        out_specs=pl.BlockSpec((_BN, _H), lambda i: (i, 0)),
        out_shape=jax.ShapeDtypeStruct((_NP, _H), _F32),
        compiler_params=pltpu.CompilerParams(
            dimension_semantics=("parallel",)),
    )(pts_pad, wp, bp, w0, b0, w1, b1, ws)


def _stage_b_body(net_ref, pl_ref, w0a_ref, w0b_ref, b0_ref, w1_ref, b1_ref,
                  wsa_ref, wsb_ref, out_ref):
    net = net_ref[...]
    pld = pl_ref[...]
    h = (_dot(jax.nn.relu(net), w0a_ref[...])
         + _dot(jax.nn.relu(pld), w0b_ref[...]) + b0_ref[...])
    dx = _dot(jax.nn.relu(h), w1_ref[...]) + b1_ref[...]
    out_ref[...] = _dot(net, wsa_ref[...]) + _dot(pld, wsb_ref[...]) + dx


def _stage_bc_body(net_ref, pl_ref, w0a_ref, w0b_ref, b0_ref, w1_ref, b1_ref,
                   wsa_ref, wsb_ref, wc_ref, bc_ref, out_ref):
    net = net_ref[...]
    pld = pl_ref[...]
    h = (_dot(jax.nn.relu(net), w0a_ref[...])
         + _dot(jax.nn.relu(pld), w0b_ref[...]) + b0_ref[...])
    dx = _dot(jax.nn.relu(h), w1_ref[...]) + b1_ref[...]
    net2 = _dot(net, wsa_ref[...]) + _dot(pld, wsb_ref[...]) + dx
    out_ref[...] = _dot(net2, wc_ref[...]) + bc_ref[...]


def _stage_b(net, pooled, w0a, w0b, b0, w1, b1, wsa, wsb, wc=None, bc=None):
    nblk = _NP // _BN
    last = wc is not None

    def spec2(a):
        return pl.BlockSpec(a.shape, lambda i: (0, 0))

    args = [net, pooled, w0a, w0b, b0, w1, b1, wsa, wsb]
    if last:
        args += [wc, bc]
    in_specs = [pl.BlockSpec((_BN, _H), lambda i: (i, 0)),
                pl.BlockSpec((_BN, _H), lambda i: (i, 0))]
    in_specs += [spec2(a) for a in args[2:]]
    return pl.pallas_call(
        _stage_bc_body if last else _stage_b_body,
        grid=(nblk,),
        in_specs=in_specs,
        out_specs=pl.BlockSpec((_BN, _CD if last else _H), lambda i: (i, 0)),
        out_shape=jax.ShapeDtypeStruct((_NP, _CD if last else _H), _F32),
        compiler_params=pltpu.CompilerParams(
            dimension_semantics=("parallel",)),
    )(*args)


def _transpose_body(in_ref, cnt_ref, out_ref):
    out_ref[...] = in_ref[...].T / jnp.maximum(cnt_ref[...], 1.0)


def _transpose_vc(fea_vc, cnt):
    return pl.pallas_call(
        _transpose_body,
        grid=(_V // _CD,),
        in_specs=[pl.BlockSpec((_CD, _CD), lambda i: (i, 0)),
                  pl.BlockSpec((1, _CD), lambda i: (0, i))],
        out_specs=pl.BlockSpec((_CD, _CD), lambda i: (0, i)),
        out_shape=jax.ShapeDtypeStruct((_CD, _V), _F32),
        compiler_params=pltpu.CompilerParams(
            dimension_semantics=("arbitrary",)),
    )(fea_vc, cnt)


_MESH = plsc.VectorSubcoreMesh(core_axis_name="c", subcore_axis_name="s")
_SC_PARAMS = pltpu.CompilerParams(needs_layout_passes=False)


def _wid():
    return lax.axis_index("s") * 2 + lax.axis_index("c")


_LCAP = 512
_STAG = 4096


@functools.partial(
    pl.kernel, mesh=_MESH, compiler_params=_SC_PARAMS,
    out_type=(jax.ShapeDtypeStruct((_NREG * _PCAP,), _I32),
              jax.ShapeDtypeStruct((_NREG * _PCAP,), _I32),
              jax.ShapeDtypeStruct((_NREG, 16), _I32)),
    scratch_types=[
        pltpu.VMEM((_SCH,), _I32),
        pltpu.VMEM((16, _LCAP), _I32), pltpu.VMEM((16, _LCAP), _I32),
        pltpu.VMEM((16, _LCAP), _I32), pltpu.VMEM((16, _LCAP), _I32),
        pltpu.VMEM((_STAG + 16,), _I32),
        pltpu.VMEM((16,), _I32),
    ],
)
def _k_pre(idx_hbm, pidl_hbm, voxl_hbm, cnts_hbm,
           idx_v, ap0, av0, ap1, av1, stag, cntv):
    wid = _wid()
    zi = jnp.zeros((16,), _I32)
    lanes = lax.iota(_I32, 16)
    accs = ((ap0, av0), (ap1, av1))

    for buf in (ap0, av0, ap1, av1):
        def zb(e, _, buf=buf):
            buf[e // (_LCAP // 16), pl.ds((e % (_LCAP // 16)) * 16, 16)] = zi
            return 0
        lax.fori_loop(0, 16 * (_LCAP // 16), zb, 0)

    def zs(e, _):
        stag[pl.ds(e * 16, 16)] = zi
        return 0
    lax.fori_loop(0, _STAG // 16, zs, 0)

    for b in range(_B):
        lo0 = wid * _NR * _VR
        reg0 = (b * _NW + wid) * _NR

        def chunk_body(ci, st):
            cur0, cur1 = st
            base = pl.multiple_of(ci * _SCH, 8)
            pltpu.sync_copy(idx_hbm.at[pl.ds(b * _N + base, _SCH)], idx_v)

            def vec_body(kk, st2):
                c0, c1 = st2
                v = idx_v[pl.ds(kk * 16, 16)]
                pid = base + kk * 16 + lanes
                curs = [c0, c1]
                for r in range(_NR):
                    lo = lo0 + r * _VR
                    m = (v >= lo) & (v < lo + _VR) & (curs[r] < _LCAP)
                    plsc.store_scatter(accs[r][0], [lanes, curs[r]], pid,
                                       mask=m)
                    plsc.store_scatter(accs[r][1], [lanes, curs[r]], v,
                                       mask=m)
                    curs[r] = curs[r] + m.astype(_I32)
                return (curs[0], curs[1])

            return lax.fori_loop(0, _SCH // 16, vec_body, (cur0, cur1))

        cur0, cur1 = lax.fori_loop(0, _N // _SCH, chunk_body, (zi, zi))

        for r in range(_NR):
            curs = (cur0, cur1)[r]
            rbase = (reg0 + r) * _PCAP
            ap, av = accs[r]
            for src, dsthbm in ((ap, pidl_hbm), (av, voxl_hbm)):
                def lane_copy(total, lane, src=src):
                    cl = curs[lane]
                    nv = (cl + 15) // 16

                    @pl.when(total < _STAG - _LCAP - 16)
                    def _cp(total=total, lane=lane, src=src, nv=nv):
                        def cp(t, _):
                            stag[pl.ds(total + t * 16, 16)] = (
                                src[lane, pl.ds(t * 16, 16)])
                            return 0
                        lax.fori_loop(0, nv, cp, 0)
                    return total + cl

                total = 0
                for lane in range(16):
                    total = lane_copy(total, lane)
                off = pl.multiple_of(rbase, 8)
                off2 = pl.multiple_of(rbase + _FB, 8)
                pltpu.sync_copy(stag.at[pl.ds(0, _FB)],
                                dsthbm.at[pl.ds(off, _FB)])
                pltpu.sync_copy(stag.at[pl.ds(_FB, _FB)],
                                dsthbm.at[pl.ds(off2, _FB)])
            cntv[...] = jnp.minimum(jnp.full((16,), total, _I32),
                                    _STAG - _LCAP)
            pltpu.sync_copy(cntv, cnts_hbm.at[reg0 + r])


def _make_permute(b):
    @functools.partial(
        pl.kernel, mesh=_MESH, compiler_params=_SC_PARAMS,
        out_type=jax.ShapeDtypeStruct((_NP, _H), _F32),
        scratch_types=[
            pltpu.VMEM((_STAG + 16,), _I32),
            pltpu.VMEM((_GCH, _H), _F32),
            pltpu.VMEM((16,), _I32),
            pltpu.VMEM((16,), _I32),
            pltpu.SemaphoreType.DMA,
        ],
    )
    def k(pts_hbm, pidl_hbm, cnts_hbm, offs_hbm, out_hbm,
          pidall, rows_v, cntv, offv, sem):
        wid = _wid()

        def region_body(r, _):
            reg = (b * _NW + wid) * _NR + r
            lreg = wid * _NR + r
            rbase = reg * _PCAP
            pltpu.sync_copy(cnts_hbm.at[reg], cntv)
            pltpu.sync_copy(offs_hbm.at[lreg], offv)
            cnt = cntv[...][0]
            off = offv[...][0]
            lbase = pl.multiple_of(rbase, 8)
            pltpu.sync_copy(pidl_hbm.at[pl.ds(lbase, _STAG)],
                            pidall.at[pl.ds(0, _STAG)])
            ng = (cnt + _GCH - 1) // _GCH

            def body(g, _):
                gpos = pl.multiple_of(g * _GCH, 8)
                pltpu.async_copy(
                    pts_hbm.at[pidall.at[pl.ds(gpos, _GCH)]],
                    rows_v, sem).wait()
                dst = pl.multiple_of(off + gpos, 8)
                pltpu.sync_copy(rows_v, out_hbm.at[pl.ds(dst, _GCH)])
                return 0

            lax.fori_loop(0, ng, body, 0)
            return 0

        lax.fori_loop(0, _NR, region_body, 0)

    return k


_k_permute = [_make_permute(b) for b in range(_B)]


def _round_scratch():
    return [
        pltpu.VMEM((_STAG + 16,), _I32),
        pltpu.VMEM((_STAG + 16,), _I32),
        pltpu.VMEM((1, _GCH), _I32),
        pltpu.VMEM((1, _GCH, _H), _F32),
        pltpu.VMEM((_GCH, _H), _F32),
        pltpu.VMEM((_VR + 16, _H), _F32),
        pltpu.VMEM((_VR + 16,), _F32),
        pltpu.VMEM((16,), _I32),
        pltpu.SemaphoreType.DMA,
        pltpu.SemaphoreType.DMA,
    ]


def _make_round(is_max, b):
    out_type = jax.ShapeDtypeStruct((_NP, _H), _F32)
    if not is_max:
        out_type = (jax.ShapeDtypeStruct((_V, _H), _F32),
                    jax.ShapeDtypeStruct((_V,), _F32))

    @functools.partial(
        pl.kernel, mesh=_MESH, compiler_params=_SC_PARAMS,
        out_type=out_type, scratch_types=_round_scratch(),
    )
    def k(net_hbm, voxl_hbm, cnts_hbm, offs_hbm, *rest):
        if is_max:
            (out_hbm, pidall, voxall, pidc, rows_v, stag_v, tab_v, cvox_v,
             cntv, sem, sem2) = rest
        else:
            (seg_hbm, cfull_hbm, pidall, voxall, pidc, rows_v, stag_v,
             tab_v, cvox_v, cntv, sem, sem2) = rest
        wid = _wid()
        zf = jnp.zeros((16,), _F32)
        neg = jnp.full((16,), _NEG, _F32)
        lane0 = lax.iota(_I32, 16) == 0
        lanes = lax.iota(_I32, 16)

        def region_body(r, _):
                reg = (b * _NW + wid) * _NR + r
                rbase = reg * _PCAP
                lo = (wid * _NR + r) * _VR
                lreg = wid * _NR + r
                pltpu.sync_copy(cnts_hbm.at[reg], cntv)
                cnt = cntv[...][0]
                pltpu.sync_copy(offs_hbm.at[lreg], pidc.at[0, pl.ds(0, 16)])
                off = pidc[0, pl.ds(0, 16)][0]
                lbase = pl.multiple_of(rbase, 8)
                pltpu.sync_copy(voxl_hbm.at[pl.ds(lbase, _STAG)],
                                voxall.at[pl.ds(0, _STAG)])

                def ztab(rr, _):
                    for gf in range(_H // 16):
                        tab_v[rr, pl.ds(gf * 16, 16)] = (
                            neg if is_max else zf)
                    return 0
                lax.fori_loop(0, _VR + 1, ztab, 0)

                if not is_max:
                    def zc(i, _):
                        cvox_v[pl.ds(i * 16, 16)] = zf
                        return 0
                    lax.fori_loop(0, (_VR + 16) // 16, zc, 0)

                ng = (cnt + _GCH - 1) // _GCH

                def acc_body(g, _):
                    gpos = pl.multiple_of(g * _GCH, 8)
                    src = pl.multiple_of(off + gpos, 8)
                    pltpu.sync_copy(net_hbm.at[pl.ds(src, _GCH)],
                                    rows_v.at[0])
                    buf = 0

                    def grp(q, _):
                        voxg = voxall[pl.ds(gpos + q * 16, 16)]
                        posg = gpos + q * 16 + lanes
                        rlv = jnp.where(posg < cnt, voxg - lo, _VR)
                        for j in range(16):
                            rl = rlv[j]
                            i = q * 16 + j
                            sls = [pl.ds(gf * 16, 16)
                                   for gf in range(_H // 16)]
                            ts = [tab_v[rl, sl] for sl in sls]
                            xs = [rows_v[buf, i, sl] for sl in sls]
                            if is_max:
                                rs = [jnp.maximum(t, x)
                                      for t, x in zip(ts, xs)]
                            else:
                                rs = [t + x for t, x in zip(ts, xs)]
                            for sl, rv in zip(sls, rs):
                                tab_v[rl, sl] = rv
                            if not is_max:
                                cv = cvox_v[pl.ds(rl, 16)]
                                cvox_v[pl.ds(rl, 16)] = jnp.where(
                                    lane0, cv + 1.0, cv)
                        return 0

                    lax.fori_loop(0, _GCH // 16, grp, 0)
                    return 0

                lax.fori_loop(0, ng, acc_body, 0)

                if is_max:
                    def exp_body(g, _):
                        gpos = pl.multiple_of(g * _GCH, 8)

                        for q in range(_GCH // 16):
                            voxg = voxall[pl.ds(gpos + q * 16, 16)]
                            posg = gpos + q * 16 + lanes
                            rlv = jnp.where(posg < cnt, voxg - lo, _VR)
                            for j in range(16):
                                rl = rlv[j]
                                i = q * 16 + j
                                sls = [pl.ds(gf * 16, 16)
                                       for gf in range(_H // 16)]
                                ts = [tab_v[rl, sl] for sl in sls]
                                for sl, tv in zip(sls, ts):
                                    stag_v[i, sl] = tv
                        dst = pl.multiple_of(off + gpos, 8)
                        pltpu.sync_copy(stag_v,
                                        out_hbm.at[pl.ds(dst, _GCH)])
                        return 0

                    lax.fori_loop(0, ng, exp_body, 0)
                else:
                    pltpu.sync_copy(tab_v.at[pl.ds(0, _VR)],
                                    seg_hbm.at[pl.ds(lo, _VR)])
                    pltpu.sync_copy(
                        cvox_v.at[pl.ds(0, _VR)],
                        cfull_hbm.at[pl.ds(lo, _VR)])
                return 0

        lax.fori_loop(0, _NR, region_body, 0)

    return k


_k_round_max = [_make_round(True, b) for b in range(_B)]
_k_mean = [_make_round(False, b) for b in range(_B)]


def kernel(points, fc_pos_W, fc_pos_b, W0s, b0s, W1s, b1s, Wss, fc_c_W,
           fc_c_b, index):
    idx_flat = index[:, 0, :].reshape(-1)

    pts = jnp.pad(points, ((0, 0), (0, 0), (0, _H - _D)))
    wp = jnp.pad(fc_pos_W, ((0, _H - _D), (0, 0)))

    pidl, voxl, cnts = _k_pre(idx_flat)

    counts = cnts[:, 0]
    offs_rows = []
    for b in range(_B):
        cb = counts[b * _NW * _NR:(b + 1) * _NW * _NR]
        slots = ((cb + _GCH - 1) // _GCH) * _GCH
        ob = jnp.concatenate([jnp.zeros((1,), _I32),
                              jnp.cumsum(slots)[:-1].astype(_I32)])
        offs_rows.append(jnp.broadcast_to(ob[:, None],
                                          (_NW * _NR, 16)).astype(_I32))

    ptss = [_k_permute[b](pts[b], pidl, cnts, offs_rows[b])
            for b in range(_B)]

    nets = [_stage_a(ptss[b], wp, fc_pos_b.reshape(1, -1), W0s[0],
                     b0s[0].reshape(1, -1), W1s[0], b1s[0].reshape(1, -1),
                     Wss[0])
            for b in range(_B)]

    for i in range(1, _NB):
        last = i == _NB - 1
        pooleds = [_k_round_max[b](nets[b], voxl, cnts, offs_rows[b])
                   for b in range(_B)]
        nets = [_stage_b(nets[b], pooleds[b],
                         W0s[i][:_H], W0s[i][_H:], b0s[i].reshape(1, -1),
                         W1s[i], b1s[i].reshape(1, -1),
                         Wss[i][:_H], Wss[i][_H:],
                         wc=fc_c_W if last else None,
                         bc=fc_c_b.reshape(1, -1) if last else None)
                for b in range(_B)]

    feas = [_k_mean[b](nets[b], voxl, cnts, offs_rows[b])
            for b in range(_B)]
    outs = [_transpose_vc(feas[b][0], feas[b][1].reshape(1, _V))
            for b in range(_B)]
    return jnp.stack(outs).reshape(_B, _CD, _R, _R, _R)


# speedup vs baseline: 1.1053x; 1.1053x over previous
import functools

import jax
import jax.numpy as jnp
from jax import lax
from jax.experimental import pallas as pl
from jax.experimental.pallas import tpu as pltpu
from jax.experimental.pallas import tpu_sc as plsc

_B, _N, _D = 2, 100000, 3
_H = 128
_CD = 128
_R = 32
_V = _R ** 3
_NB = 5

_NW = 32
_BN = 2000
_VR = 512
_NR = _V // (_VR * _NW)
_NREG = _B * _NW * _NR
_SCH = 2000
_GCH = 112
_FB = 2048
_ACC = 4096
_PCAP = 8192
_NP = 118000

_F32 = jnp.float32
_I32 = jnp.int32

_NEG = -3.0e38


def _dot(a, b):
    return jnp.dot(a, b, preferred_element_type=_F32)


def _stage_a_body(x_ref, wp_ref, bp_ref, w0_ref, b0_ref, w1_ref, b1_ref,
                  ws_ref, out_ref):
    x = x_ref[...]
    x1 = _dot(x, wp_ref[...]) + bp_ref[...]
    h = _dot(jax.nn.relu(x1), w0_ref[...]) + b0_ref[...]
    dx = _dot(jax.nn.relu(h), w1_ref[...]) + b1_ref[...]
    out_ref[...] = _dot(x1, ws_ref[...]) + dx


def _stage_a(pts_pad, wp, bp, w0, b0, w1, b1, ws):
    nblk = _NP // _BN

    def spec2(a):
        return pl.BlockSpec(a.shape, lambda i: (0, 0))

    return pl.pallas_call(
        _stage_a_body,
        grid=(nblk,),
        in_specs=[
            pl.BlockSpec((_BN, _H), lambda i: (i, 0)),
            spec2(wp), spec2(bp), spec2(w0), spec2(b0), spec2(w1), spec2(b1),
            spec2(ws),
        ],
        out_specs=pl.BlockSpec((_BN, _H), lambda i: (i, 0)),
        out_shape=jax.ShapeDtypeStruct((_NP, _H), _F32),
        compiler_params=pltpu.CompilerParams(
            dimension_semantics=("parallel",)),
    )(pts_pad, wp, bp, w0, b0, w1, b1, ws)


def _stage_b_body(net_ref, pl_ref, w0a_ref, w0b_ref, b0_ref, w1_ref, b1_ref,
                  wsa_ref, wsb_ref, out_ref):
    net = net_ref[...]
    pld = pl_ref[...]
    h = (_dot(jax.nn.relu(net), w0a_ref[...])
         + _dot(jax.nn.relu(pld), w0b_ref[...]) + b0_ref[...])
    dx = _dot(jax.nn.relu(h), w1_ref[...]) + b1_ref[...]
    out_ref[...] = _dot(net, wsa_ref[...]) + _dot(pld, wsb_ref[...]) + dx


def _stage_bc_body(net_ref, pl_ref, w0a_ref, w0b_ref, b0_ref, w1_ref, b1_ref,
                   wsa_ref, wsb_ref, wc_ref, bc_ref, out_ref):
    net = net_ref[...]
    pld = pl_ref[...]
    h = (_dot(jax.nn.relu(net), w0a_ref[...])
         + _dot(jax.nn.relu(pld), w0b_ref[...]) + b0_ref[...])
    dx = _dot(jax.nn.relu(h), w1_ref[...]) + b1_ref[...]
    net2 = _dot(net, wsa_ref[...]) + _dot(pld, wsb_ref[...]) + dx
    out_ref[...] = _dot(net2, wc_ref[...]) + bc_ref[...]


def _stage_b(net, pooled, w0a, w0b, b0, w1, b1, wsa, wsb, wc=None, bc=None):
    nblk = _NP // _BN
    last = wc is not None

    def spec2(a):
        return pl.BlockSpec(a.shape, lambda i: (0, 0))

    args = [net, pooled, w0a, w0b, b0, w1, b1, wsa, wsb]
    if last:
        args += [wc, bc]
    in_specs = [pl.BlockSpec((_BN, _H), lambda i: (i, 0)),
                pl.BlockSpec((_BN, _H), lambda i: (i, 0))]
    in_specs += [spec2(a) for a in args[2:]]
    return pl.pallas_call(
        _stage_bc_body if last else _stage_b_body,
        grid=(nblk,),
        in_specs=in_specs,
        out_specs=pl.BlockSpec((_BN, _CD if last else _H), lambda i: (i, 0)),
        out_shape=jax.ShapeDtypeStruct((_NP, _CD if last else _H), _F32),
        compiler_params=pltpu.CompilerParams(
            dimension_semantics=("parallel",)),
    )(*args)


def _transpose_body(in_ref, cnt_ref, out_ref):
    out_ref[...] = in_ref[...].T / jnp.maximum(cnt_ref[...], 1.0)


def _transpose_vc(fea_vc, cnt):
    return pl.pallas_call(
        _transpose_body,
        grid=(_V // _CD,),
        in_specs=[pl.BlockSpec((_CD, _CD), lambda i: (i, 0)),
                  pl.BlockSpec((1, _CD), lambda i: (0, i))],
        out_specs=pl.BlockSpec((_CD, _CD), lambda i: (0, i)),
        out_shape=jax.ShapeDtypeStruct((_CD, _V), _F32),
        compiler_params=pltpu.CompilerParams(
            dimension_semantics=("arbitrary",)),
    )(fea_vc, cnt)


_MESH = plsc.VectorSubcoreMesh(core_axis_name="c", subcore_axis_name="s")
_SC_PARAMS = pltpu.CompilerParams(needs_layout_passes=False)


def _wid():
    return lax.axis_index("s") * 2 + lax.axis_index("c")


_LCAP = 512
_STAG = 4096


@functools.partial(
    pl.kernel, mesh=_MESH, compiler_params=_SC_PARAMS,
    out_type=(jax.ShapeDtypeStruct((_NREG * _PCAP,), _I32),
              jax.ShapeDtypeStruct((_NREG * _PCAP,), _I32),
              jax.ShapeDtypeStruct((_NREG, 16), _I32)),
    scratch_types=[
        pltpu.VMEM((_SCH,), _I32),
        pltpu.VMEM((16, _LCAP), _I32), pltpu.VMEM((16, _LCAP), _I32),
        pltpu.VMEM((16, _LCAP), _I32), pltpu.VMEM((16, _LCAP), _I32),
        pltpu.VMEM((_STAG + 16,), _I32),
        pltpu.VMEM((16,), _I32),
    ],
)
def _k_pre(idx_hbm, pidl_hbm, voxl_hbm, cnts_hbm,
           idx_v, ap0, av0, ap1, av1, stag, cntv):
    wid = _wid()
    zi = jnp.zeros((16,), _I32)
    lanes = lax.iota(_I32, 16)
    accs = ((ap0, av0), (ap1, av1))

    for buf in (ap0, av0, ap1, av1):
        def zb(e, _, buf=buf):
            buf[e // (_LCAP // 16), pl.ds((e % (_LCAP // 16)) * 16, 16)] = zi
            return 0
        lax.fori_loop(0, 16 * (_LCAP // 16), zb, 0)

    def zs(e, _):
        stag[pl.ds(e * 16, 16)] = zi
        return 0
    lax.fori_loop(0, _STAG // 16, zs, 0)

    for b in range(_B):
        lo0 = wid * _NR * _VR
        reg0 = (b * _NW + wid) * _NR

        def chunk_body(ci, st):
            cur0, cur1 = st
            base = pl.multiple_of(ci * _SCH, 8)
            pltpu.sync_copy(idx_hbm.at[pl.ds(b * _N + base, _SCH)], idx_v)

            def vec_body(kk, st2):
                c0, c1 = st2
                v = idx_v[pl.ds(kk * 16, 16)]
                pid = base + kk * 16 + lanes
                curs = [c0, c1]
                for r in range(_NR):
                    lo = lo0 + r * _VR
                    m = (v >= lo) & (v < lo + _VR) & (curs[r] < _LCAP)
                    plsc.store_scatter(accs[r][0], [lanes, curs[r]], pid,
                                       mask=m)
                    plsc.store_scatter(accs[r][1], [lanes, curs[r]], v,
                                       mask=m)
                    curs[r] = curs[r] + m.astype(_I32)
                return (curs[0], curs[1])

            return lax.fori_loop(0, _SCH // 16, vec_body, (cur0, cur1))

        cur0, cur1 = lax.fori_loop(0, _N // _SCH, chunk_body, (zi, zi))

        for r in range(_NR):
            curs = (cur0, cur1)[r]
            rbase = (reg0 + r) * _PCAP
            ap, av = accs[r]
            for src, dsthbm in ((ap, pidl_hbm), (av, voxl_hbm)):
                def lane_copy(total, lane, src=src):
                    cl = curs[lane]
                    nv = (cl + 15) // 16

                    @pl.when(total < _STAG - _LCAP - 16)
                    def _cp(total=total, lane=lane, src=src, nv=nv):
                        def cp(t, _):
                            stag[pl.ds(total + t * 16, 16)] = (
                                src[lane, pl.ds(t * 16, 16)])
                            return 0
                        lax.fori_loop(0, nv, cp, 0)
                    return total + cl

                total = 0
                for lane in range(16):
                    total = lane_copy(total, lane)
                off = pl.multiple_of(rbase, 8)
                off2 = pl.multiple_of(rbase + _FB, 8)
                pltpu.sync_copy(stag.at[pl.ds(0, _FB)],
                                dsthbm.at[pl.ds(off, _FB)])
                pltpu.sync_copy(stag.at[pl.ds(_FB, _FB)],
                                dsthbm.at[pl.ds(off2, _FB)])
            cntv[...] = jnp.minimum(jnp.full((16,), total, _I32),
                                    _STAG - _LCAP)
            pltpu.sync_copy(cntv, cnts_hbm.at[reg0 + r])


def _make_permute(b):
    @functools.partial(
        pl.kernel, mesh=_MESH, compiler_params=_SC_PARAMS,
        out_type=jax.ShapeDtypeStruct((_NP, _H), _F32),
        scratch_types=[
            pltpu.VMEM((_STAG + 16,), _I32),
            pltpu.VMEM((_GCH, _H), _F32),
            pltpu.VMEM((16,), _I32),
            pltpu.VMEM((16,), _I32),
            pltpu.SemaphoreType.DMA,
        ],
    )
    def k(pts_hbm, pidl_hbm, cnts_hbm, offs_hbm, out_hbm,
          pidall, rows_v, cntv, offv, sem):
        wid = _wid()

        def region_body(r, _):
            reg = (b * _NW + wid) * _NR + r
            lreg = wid * _NR + r
            rbase = reg * _PCAP
            pltpu.sync_copy(cnts_hbm.at[reg], cntv)
            pltpu.sync_copy(offs_hbm.at[lreg], offv)
            cnt = cntv[...][0]
            off = offv[...][0]
            lbase = pl.multiple_of(rbase, 8)
            pltpu.sync_copy(pidl_hbm.at[pl.ds(lbase, _STAG)],
                            pidall.at[pl.ds(0, _STAG)])
            ng = (cnt + _GCH - 1) // _GCH

            def body(g, _):
                gpos = pl.multiple_of(g * _GCH, 8)
                pltpu.async_copy(
                    pts_hbm.at[pidall.at[pl.ds(gpos, _GCH)]],
                    rows_v, sem).wait()
                dst = pl.multiple_of(off + gpos, 8)
                pltpu.sync_copy(rows_v, out_hbm.at[pl.ds(dst, _GCH)])
                return 0

            lax.fori_loop(0, ng, body, 0)
            return 0

        lax.fori_loop(0, _NR, region_body, 0)

    return k


_k_permute = [_make_permute(b) for b in range(_B)]


def _round_scratch():
    return [
        pltpu.VMEM((_STAG + 16,), _I32),
        pltpu.VMEM((1, _GCH), _I32),
        pltpu.VMEM((2, _GCH, _H), _F32),
        pltpu.VMEM((2, _GCH, _H), _F32),
        pltpu.VMEM((_VR + 16, _H), _F32),
        pltpu.VMEM((_VR + 16,), _F32),
        pltpu.VMEM((16,), _I32),
        pltpu.SemaphoreType.DMA,
        pltpu.SemaphoreType.DMA,
    ]


def _make_round(is_max, b):
    out_type = jax.ShapeDtypeStruct((_NP, _H), _F32)
    if not is_max:
        out_type = (jax.ShapeDtypeStruct((_V, _H), _F32),
                    jax.ShapeDtypeStruct((_V,), _F32))

    @functools.partial(
        pl.kernel, mesh=_MESH, compiler_params=_SC_PARAMS,
        out_type=out_type, scratch_types=_round_scratch(),
    )
    def k(net_hbm, voxl_hbm, cnts_hbm, offs_hbm, *rest):
        if is_max:
            (out_hbm, voxall, pidc, rows_v, stag_v, tab_v, cvox_v,
             cntv, sem, sem2) = rest
        else:
            (seg_hbm, cfull_hbm, voxall, pidc, rows_v, stag_v,
             tab_v, cvox_v, cntv, sem, sem2) = rest
        wid = _wid()
        zf = jnp.zeros((16,), _F32)
        neg = jnp.full((16,), _NEG, _F32)
        lane0 = lax.iota(_I32, 16) == 0
        lanes = lax.iota(_I32, 16)

        def region_body(r, _):
                reg = (b * _NW + wid) * _NR + r
                rbase = reg * _PCAP
                lo = (wid * _NR + r) * _VR
                lreg = wid * _NR + r
                pltpu.sync_copy(cnts_hbm.at[reg], cntv)
                cnt = cntv[...][0]
                pltpu.sync_copy(offs_hbm.at[lreg], pidc.at[0, pl.ds(0, 16)])
                off = pidc[0, pl.ds(0, 16)][0]
                lbase = pl.multiple_of(rbase, 8)
                pltpu.sync_copy(voxl_hbm.at[pl.ds(lbase, _STAG)],
                                voxall.at[pl.ds(0, _STAG)])

                def ztab(rr, _):
                    for gf in range(_H // 16):
                        tab_v[rr, pl.ds(gf * 16, 16)] = (
                            neg if is_max else zf)
                    return 0
                lax.fori_loop(0, _VR + 1, ztab, 0)

                if not is_max:
                    def zc(i, _):
                        cvox_v[pl.ds(i * 16, 16)] = zf
                        return 0
                    lax.fori_loop(0, (_VR + 16) // 16, zc, 0)

                ng = (cnt + _GCH - 1) // _GCH
                nge = (ng + 1) // 2

                def start_load(g, buf):
                    src = pl.multiple_of(off + g * _GCH, 8)
                    return pltpu.async_copy(net_hbm.at[pl.ds(src, _GCH)],
                                            rows_v.at[buf], sem)

                def process(g, buf):
                    gpos = pl.multiple_of(g * _GCH, 8)

                    def grp(q, _):
                        voxg = voxall[pl.ds(gpos + q * 16, 16)]
                        posg = gpos + q * 16 + lanes
                        rlv = jnp.where(posg < cnt, voxg - lo, _VR)
                        for j in range(16):
                            rl = rlv[j]
                            i = q * 16 + j
                            sls = [pl.ds(gf * 16, 16)
                                   for gf in range(_H // 16)]
                            ts = [tab_v[rl, sl] for sl in sls]
                            xs = [rows_v[buf, i, sl] for sl in sls]
                            if is_max:
                                rs = [jnp.maximum(t, x)
                                      for t, x in zip(ts, xs)]
                            else:
                                rs = [t + x for t, x in zip(ts, xs)]
                            for sl, rv in zip(sls, rs):
                                tab_v[rl, sl] = rv
                            if not is_max:
                                cv = cvox_v[pl.ds(rl, 16)]
                                cvox_v[pl.ds(rl, 16)] = jnp.where(
                                    lane0, cv + 1.0, cv)
                        return 0

                    lax.fori_loop(0, _GCH // 16, grp, 0)

                start_load(0, 0).wait()

                def acc_pair(gp, _):
                    cpb = start_load(2 * gp + 1, 1)
                    process(2 * gp, 0)
                    cpa = start_load(2 * gp + 2, 0)
                    cpb.wait()
                    process(2 * gp + 1, 1)
                    cpa.wait()
                    return 0

                lax.fori_loop(0, nge, acc_pair, 0)

                if is_max:
                    def fill(g, buf):
                        gpos = pl.multiple_of(g * _GCH, 8)
                        for q in range(_GCH // 16):
                            voxg = voxall[pl.ds(gpos + q * 16, 16)]
                            posg = gpos + q * 16 + lanes
                            rlv = jnp.where(posg < cnt, voxg - lo, _VR)
                            for j in range(16):
                                rl = rlv[j]
                                i = q * 16 + j
                                sls = [pl.ds(gf * 16, 16)
                                       for gf in range(_H // 16)]
                                ts = [tab_v[rl, sl] for sl in sls]
                                for sl, tv in zip(sls, ts):
                                    stag_v[buf, i, sl] = tv

                    def put(g, buf):
                        dst = pl.multiple_of(off + g * _GCH, 8)
                        return pltpu.async_copy(
                            stag_v.at[buf], out_hbm.at[pl.ds(dst, _GCH)],
                            sem2)

                    def exp_pair(gp, _):
                        fill(2 * gp, 0)
                        cpa = put(2 * gp, 0)
                        fill(2 * gp + 1, 1)
                        cpb = put(2 * gp + 1, 1)
                        cpa.wait()
                        cpb.wait()
                        return 0

                    lax.fori_loop(0, nge, exp_pair, 0)
                else:
                    pltpu.sync_copy(tab_v.at[pl.ds(0, _VR)],
                                    seg_hbm.at[pl.ds(lo, _VR)])
                    pltpu.sync_copy(
                        cvox_v.at[pl.ds(0, _VR)],
                        cfull_hbm.at[pl.ds(lo, _VR)])
                return 0

        lax.fori_loop(0, _NR, region_body, 0)

    return k


_k_round_max = [_make_round(True, b) for b in range(_B)]
_k_mean = [_make_round(False, b) for b in range(_B)]


def kernel(points, fc_pos_W, fc_pos_b, W0s, b0s, W1s, b1s, Wss, fc_c_W,
           fc_c_b, index):
    idx_flat = index[:, 0, :].reshape(-1)

    pts = jnp.pad(points, ((0, 0), (0, 0), (0, _H - _D)))
    wp = jnp.pad(fc_pos_W, ((0, _H - _D), (0, 0)))

    pidl, voxl, cnts = _k_pre(idx_flat)

    counts = cnts[:, 0]
    offs_rows = []
    for b in range(_B):
        cb = counts[b * _NW * _NR:(b + 1) * _NW * _NR]
        slots = ((cb + _GCH - 1) // _GCH + 1) * _GCH
        ob = jnp.concatenate([jnp.zeros((1,), _I32),
                              jnp.cumsum(slots)[:-1].astype(_I32)])
        offs_rows.append(jnp.broadcast_to(ob[:, None],
                                          (_NW * _NR, 16)).astype(_I32))

    ptss = [_k_permute[b](pts[b], pidl, cnts, offs_rows[b])
            for b in range(_B)]

    nets = [_stage_a(ptss[b], wp, fc_pos_b.reshape(1, -1), W0s[0],
                     b0s[0].reshape(1, -1), W1s[0], b1s[0].reshape(1, -1),
                     Wss[0])
            for b in range(_B)]

    for i in range(1, _NB):
        last = i == _NB - 1
        pooleds = [_k_round_max[b](nets[b], voxl, cnts, offs_rows[b])
                   for b in range(_B)]
        nets = [_stage_b(nets[b], pooleds[b],
                         W0s[i][:_H], W0s[i][_H:], b0s[i].reshape(1, -1),
                         W1s[i], b1s[i].reshape(1, -1),
                         Wss[i][:_H], Wss[i][_H:],
                         wc=fc_c_W if last else None,
                         bc=fc_c_b.reshape(1, -1) if last else None)
                for b in range(_B)]

    feas = [_k_mean[b](nets[b], voxl, cnts, offs_rows[b])
            for b in range(_B)]
    outs = [_transpose_vc(feas[b][0], feas[b][1].reshape(1, _V))
            for b in range(_B)]
    return jnp.stack(outs).reshape(_B, _CD, _R, _R, _R)
